# Initial kernel scaffold; baseline (speedup 1.0000x reference)
import jax
import jax.numpy as jnp
from jax.experimental import pallas as pl


def kernel(pcd):
    raise NotImplementedError("write your pallas kernel here")


# speedup vs baseline: 9.7820x; 9.7820x over previous
import functools

import jax
import jax.numpy as jnp
from jax import lax
from jax.experimental import pallas as pl

N = 2048
G = 2048
K = 16
G_TILE = 512


def _knn_body(pcd_ref, pfull_ref, xt_ref, out_ref):
    ct = pcd_ref[0]
    pf = pfull_ref[0]
    xt = xt_ref[0]
    sq = jnp.sum(xt * xt, axis=0, keepdims=True)
    sqg = jnp.sum(ct * ct, axis=1, keepdims=True)
    dots = jax.lax.dot_general(ct, pf, (((1,), (1,)), ((), ())),
                               preferred_element_type=jnp.float32)
    d2 = (sqg + sq) - 2.0 * dots
    ii = lax.broadcasted_iota(jnp.int32, (G_TILE, N), 1)
    nbs = []
    for _ in range(K):
        m = jnp.min(d2, axis=1, keepdims=True)
        im = jnp.min(jnp.where(d2 == m, ii, N), axis=1, keepdims=True)
        onehot = ii == im
        oh = onehot.astype(jnp.float32)
        nb = jax.lax.dot_general(oh, xt, (((1,), (1,)), ((), ())),
                                 preferred_element_type=jnp.float32)
        nbs.append(nb - ct)
        d2 = jnp.where(onehot, jnp.inf, d2)
    out_ref[0] = jnp.concatenate(nbs, axis=1)


@jax.jit
def kernel(pcd):
    B = pcd.shape[0]
    xt = jnp.transpose(pcd, (0, 2, 1))
    out = pl.pallas_call(
        _knn_body,
        grid=(B, G // G_TILE),
        in_specs=[
            pl.BlockSpec((1, G_TILE, 3), lambda b, j: (b, j, 0)),
            pl.BlockSpec((1, N, 3), lambda b, j: (b, 0, 0)),
            pl.BlockSpec((1, 3, N), lambda b, j: (b, 0, 0)),
        ],
        out_specs=pl.BlockSpec((1, G_TILE, K * 3), lambda b, j: (b, j, 0)),
        out_shape=jax.ShapeDtypeStruct((B, G, K * 3), jnp.float32),
    )(pcd, pcd, xt)
    neighborhood = out.reshape(B, G, K, 3)
    return (neighborhood, pcd)
